# baseline (speedup 1.0000x reference)
import functools

import jax
import jax.numpy as jnp
from jax import lax
from jax.experimental import pallas as pl
from jax.experimental.pallas import tpu as pltpu
from jax.experimental.pallas import tpu_sc as plsc

NC = 2
NS = 16
K = 128


def _make_sc_agg(NP, W, ch0, ch1, dtype):
  SLAB = NP // NS
  D = 4
  NB = 2 * D
  CHM = max(ch0, ch1)
  mesh = plsc.VectorSubcoreMesh(core_axis_name="c", subcore_axis_name="s")

  @functools.partial(
      pl.kernel,
      out_type=jax.ShapeDtypeStruct((NC, NP, W), dtype),
      mesh=mesh,
      compiler_params=pltpu.CompilerParams(use_tc_tiling_on_sc=False),
      scratch_types=(
          [pltpu.VMEM((CHM, K), jnp.int32)] * 2
          + [pltpu.VMEM((K, W), dtype)] * NB
          + [pltpu.VMEM_SHARED((NP, W), dtype)]
          + [pltpu.SemaphoreType.DMA] * (2 * NB)
      ),
  )
  def agg(rows_hbm, src_hbm, dst_hbm, z_hbm, out_hbm, *scr):
    sidx, didx = scr[0], scr[1]
    bufs = scr[2:2 + NB]
    acc = scr[2 + NB]
    sem_g = scr[3 + NB:3 + 2 * NB]
    sem_s = scr[3 + 2 * NB:]
    c = lax.axis_index("c")
    s = lax.axis_index("s")
    pltpu.sync_copy(z_hbm, acc.at[pl.ds(s * SLAB, SLAB)])

    def start_g(t, slot):
      pltpu.async_copy(rows_hbm.at[sidx.at[t]], bufs[slot], sem_g[slot])

    def wait_g(t, slot):
      pltpu.make_async_copy(rows_hbm.at[sidx.at[t]], bufs[slot],
                            sem_g[slot]).wait()

    def start_s(t, slot):
      pltpu.async_copy(bufs[slot], acc.at[didx.at[t]], sem_s[slot], add=True)

    def wait_s(t, slot):
      pltpu.make_async_copy(bufs[slot], acc.at[didx.at[t]],
                            sem_s[slot]).wait()

    def run(ch, row0):
      pltpu.sync_copy(src_hbm.at[pl.ds(row0, ch)], sidx.at[pl.ds(0, ch)])
      pltpu.sync_copy(dst_hbm.at[pl.ds(row0, ch)], didx.at[pl.ds(0, ch)])
      plsc.subcore_barrier()
      for k in range(D):
        start_g(k, k)
      for t in range(NB):
        wait_g(t, t)
        start_s(t, t)
        if t >= D:
          wait_s(t - D, t - D)
        start_g(t + D, (t + D) % NB)

      def body(i, carry):
        base = NB * i
        for k in range(NB):
          t = base + k
          wait_g(t, k)
          start_s(t, k)
          m2 = (k + D) % NB
          wait_s(t - D, m2)
          jn = jnp.minimum(t + D, ch - 1)
          start_g(jn, m2)
        return carry

      lax.fori_loop(1, ch // NB, body, 0)
      for k in range(D):
        wait_s(ch - D + k, D + k)
      for k in range(D):
        wait_g(ch - 1, k)

    if ch0 == ch1:
      run(ch0, (s * NC + c) * ch0)
    else:
      @pl.when(c == 0)
      def _():
        run(ch0, s * ch0)

      @pl.when(c != 0)
      def _():
        run(ch1, NS * ch0 + s * ch1)

    plsc.subcore_barrier()
    pltpu.sync_copy(acc.at[pl.ds(s * SLAB, SLAB)],
                    out_hbm.at[c, pl.ds(s * SLAB, SLAB)])

  return agg


def _make_sc_degree(NP, CH):
  SLAB = NP // NS
  mesh = plsc.VectorSubcoreMesh(core_axis_name="c", subcore_axis_name="s")

  @functools.partial(
      pl.kernel,
      out_type=jax.ShapeDtypeStruct((NC, NP, 8), jnp.float32),
      mesh=mesh,
      compiler_params=pltpu.CompilerParams(use_tc_tiling_on_sc=False),
      scratch_types=[
          pltpu.VMEM((CH, K), jnp.int32),
          pltpu.VMEM((K, 8), jnp.float32),
          pltpu.VMEM_SHARED((NP, 8), jnp.float32),
          pltpu.SemaphoreType.DMA,
          pltpu.SemaphoreType.DMA,
          pltpu.SemaphoreType.DMA,
          pltpu.SemaphoreType.DMA,
      ],
  )
  def degk(dst_hbm, ones_hbm, z_hbm, out_hbm, didx, onesv, acc, *sems):
    c = lax.axis_index("c")
    s = lax.axis_index("s")
    w = s * NC + c
    pltpu.sync_copy(z_hbm, acc.at[pl.ds(s * SLAB, SLAB)])
    pltpu.sync_copy(ones_hbm, onesv)
    pltpu.sync_copy(dst_hbm.at[pl.ds(w * CH, CH)], didx)
    plsc.subcore_barrier()

    for k in range(4):
      pltpu.async_copy(onesv, acc.at[didx.at[k]], sems[k], add=True)

    def body(i, carry):
      base = 4 * i
      for k in range(4):
        j = base + k
        pltpu.make_async_copy(onesv, acc.at[didx.at[j - 4]], sems[k]).wait()
        pltpu.async_copy(onesv, acc.at[didx.at[j]], sems[k], add=True)
      return carry

    lax.fori_loop(1, CH // 4, body, 0)
    for k in range(4):
      pltpu.make_async_copy(onesv, acc.at[didx.at[CH - 4 + k]],
                            sems[k]).wait()
    plsc.subcore_barrier()
    pltpu.sync_copy(acc.at[pl.ds(s * SLAB, SLAB)],
                    out_hbm.at[c, pl.ds(s * SLAB, SLAB)])

  return degk


def _dinv_of(deg_ref):
  deg = deg_ref[0, :, 0:1] + deg_ref[1, :, 0:1] + 1.0
  return lax.rsqrt(deg)


def _tc1_body(x_ref, w_ref, deg_ref, o_ref):
  dinv = _dinv_of(deg_ref)
  o_ref[...] = (jnp.dot(x_ref[...], w_ref[...],
                        preferred_element_type=jnp.float32)
                * dinv).astype(jnp.bfloat16)


def _tc2_body(sp_ref, h_ref, deg_ref, b1_ref, w2_ref, o_ref):
  dinv = _dinv_of(deg_ref)
  s = (sp_ref[0].astype(jnp.float32) + sp_ref[1].astype(jnp.float32)
       + h_ref[...].astype(jnp.float32))
  pre = s * dinv + b1_ref[...]
  h = jnp.maximum(pre, 0.0)
  o_ref[...] = jnp.dot(h, w2_ref[...],
                       preferred_element_type=jnp.float32) * dinv


def _tc3_body(s2_ref, g_ref, deg_ref, x_ref, ws_ref, bv_ref, o_ref):
  dinv = _dinv_of(deg_ref)
  s2 = (s2_ref[0].astype(jnp.float32) + s2_ref[1].astype(jnp.float32)
        + g_ref[...].astype(jnp.float32))
  o_ref[...] = (s2 * dinv
                + jnp.dot(x_ref[...], ws_ref[...],
                          preferred_element_type=jnp.float32)
                + bv_ref[...])


def kernel(x, edge_index, W1, b1, W2, b2, Ws, bs):
  N, DIN = x.shape
  DH = W1.shape[1]
  DO = W2.shape[1]
  E = edge_index.shape[1]
  f32 = jnp.float32

  NP = -(-(N + 1) // 256) * 256
  SLAB = NP // NS
  CH = -(-E // (NC * NS * K))
  CH = -(-CH // 8) * 8
  CH0 = 64
  CH1 = 2 * CH - CH0
  R = max(NC * NS * CH, (NS + 1) * CH0 + (NS - 1) * CH1)
  EP = R * K

  src = edge_index[0]
  dst = edge_index[1]
  epad = jnp.full((EP - E,), N, dtype=jnp.int32)
  srcp = jnp.concatenate([src, epad]).reshape(R, K)
  dstp = jnp.concatenate([dst, epad]).reshape(R, K)
  xp = jnp.pad(x, ((0, NP - N), (0, 0)))
  W2p = jnp.pad(W2, ((0, 0), (0, 8 - DO)))
  Wsp = jnp.pad(Ws, ((0, 0), (0, 8 - DO)))
  bv = jnp.pad((b2 + bs).reshape(1, DO), ((0, 0), (0, 8 - DO)))
  b1r = b1.reshape(1, DH)
  ones8 = jnp.ones((K, 8), f32)
  z_dh = jnp.zeros((SLAB, DH), jnp.bfloat16)
  z_8 = jnp.zeros((SLAB, 8), f32)
  z_8b = jnp.zeros((SLAB, 8), jnp.bfloat16)

  degp = _make_sc_degree(NP, CH)(dstp, ones8, z_8)

  BM = 1024
  grid = (NP // BM,)
  h1p = pl.pallas_call(
      _tc1_body,
      grid=grid,
      in_specs=[
          pl.BlockSpec((BM, DIN), lambda i: (i, 0)),
          pl.BlockSpec((DIN, DH), lambda i: (0, 0)),
          pl.BlockSpec((NC, BM, 8), lambda i: (0, i, 0)),
      ],
      out_specs=pl.BlockSpec((BM, DH), lambda i: (i, 0)),
      out_shape=jax.ShapeDtypeStruct((NP, DH), jnp.bfloat16),
  )(xp, W1, degp)

  Sp = _make_sc_agg(NP, DH, CH0, CH1, jnp.bfloat16)(h1p, srcp, dstp, z_dh)

  gp = pl.pallas_call(
      _tc2_body,
      grid=grid,
      in_specs=[
          pl.BlockSpec((NC, BM, DH), lambda i: (0, i, 0)),
          pl.BlockSpec((BM, DH), lambda i: (i, 0)),
          pl.BlockSpec((NC, BM, 8), lambda i: (0, i, 0)),
          pl.BlockSpec((1, DH), lambda i: (0, 0)),
          pl.BlockSpec((DH, 8), lambda i: (0, 0)),
      ],
      out_specs=pl.BlockSpec((BM, 8), lambda i: (i, 0)),
      out_shape=jax.ShapeDtypeStruct((NP, 8), f32),
  )(Sp, h1p, degp, b1r, W2p)

  S2p = _make_sc_agg(NP, 8, CH, CH, f32)(gp, srcp, dstp, z_8)

  res = pl.pallas_call(
      _tc3_body,
      grid=grid,
      in_specs=[
          pl.BlockSpec((NC, BM, 8), lambda i: (0, i, 0)),
          pl.BlockSpec((BM, 8), lambda i: (i, 0)),
          pl.BlockSpec((NC, BM, 8), lambda i: (0, i, 0)),
          pl.BlockSpec((BM, DIN), lambda i: (i, 0)),
          pl.BlockSpec((DIN, 8), lambda i: (0, 0)),
          pl.BlockSpec((1, 8), lambda i: (0, 0)),
      ],
      out_specs=pl.BlockSpec((BM, 8), lambda i: (i, 0)),
      out_shape=jax.ShapeDtypeStruct((NP, 8), f32),
  )(S2p, gp, degp, xp, Wsp, bv)

  return res[:N, :DO]


# speedup vs baseline: 1.0327x; 1.0327x over previous
import functools

import jax
import jax.numpy as jnp
from jax import lax
from jax.experimental import pallas as pl
from jax.experimental.pallas import tpu as pltpu
from jax.experimental.pallas import tpu_sc as plsc

NC = 2
NS = 16
K = 128


def _make_sc_agg(NP, W, ch0, ch1, dtype):
  SLAB = NP // NS
  D = 4
  NB = 2 * D
  CHM = max(ch0, ch1)
  mesh = plsc.VectorSubcoreMesh(core_axis_name="c", subcore_axis_name="s")

  @functools.partial(
      pl.kernel,
      out_type=jax.ShapeDtypeStruct((NC, NP, W), dtype),
      mesh=mesh,
      compiler_params=pltpu.CompilerParams(use_tc_tiling_on_sc=False),
      scratch_types=(
          [pltpu.VMEM((CHM, K), jnp.int32)] * 2
          + [pltpu.VMEM((K, W), dtype)] * NB
          + [pltpu.VMEM_SHARED((NP, W), dtype)]
          + [pltpu.SemaphoreType.DMA] * (2 * NB)
      ),
  )
  def agg(rows_hbm, src_hbm, dst_hbm, z_hbm, out_hbm, *scr):
    sidx, didx = scr[0], scr[1]
    bufs = scr[2:2 + NB]
    acc = scr[2 + NB]
    sem_g = scr[3 + NB:3 + 2 * NB]
    sem_s = scr[3 + 2 * NB:]
    c = lax.axis_index("c")
    s = lax.axis_index("s")
    pltpu.sync_copy(z_hbm, acc.at[pl.ds(s * SLAB, SLAB)])

    def start_g(t, slot):
      pltpu.async_copy(rows_hbm.at[sidx.at[t]], bufs[slot], sem_g[slot])

    def wait_g(t, slot):
      pltpu.make_async_copy(rows_hbm.at[sidx.at[t]], bufs[slot],
                            sem_g[slot]).wait()

    def start_s(t, slot):
      pltpu.async_copy(bufs[slot], acc.at[didx.at[t]], sem_s[slot], add=True)

    def wait_s(t, slot):
      pltpu.make_async_copy(bufs[slot], acc.at[didx.at[t]],
                            sem_s[slot]).wait()

    def run(ch, row0):
      pltpu.sync_copy(src_hbm.at[pl.ds(row0, ch)], sidx.at[pl.ds(0, ch)])
      pltpu.sync_copy(dst_hbm.at[pl.ds(row0, ch)], didx.at[pl.ds(0, ch)])
      plsc.subcore_barrier()
      for k in range(D):
        start_g(k, k)
      for t in range(NB):
        wait_g(t, t)
        start_s(t, t)
        if t >= D:
          wait_s(t - D, t - D)
        start_g(t + D, (t + D) % NB)

      def body(i, carry):
        base = NB * i
        for k in range(NB):
          t = base + k
          wait_g(t, k)
          start_s(t, k)
          m2 = (k + D) % NB
          wait_s(t - D, m2)
          jn = jnp.minimum(t + D, ch - 1)
          start_g(jn, m2)
        return carry

      lax.fori_loop(1, ch // NB, body, 0)
      for k in range(D):
        wait_s(ch - D + k, D + k)
      for k in range(D):
        wait_g(ch - 1, k)

    if ch0 == ch1:
      run(ch0, (s * NC + c) * ch0)
    else:
      @pl.when(c == 0)
      def _():
        run(ch0, s * ch0)

      @pl.when(c != 0)
      def _():
        run(ch1, NS * ch0 + s * ch1)

    plsc.subcore_barrier()
    pltpu.sync_copy(acc.at[pl.ds(s * SLAB, SLAB)],
                    out_hbm.at[c, pl.ds(s * SLAB, SLAB)])

  return agg


def _make_sc_degree(NP, CH):
  SLAB = NP // NS
  mesh = plsc.VectorSubcoreMesh(core_axis_name="c", subcore_axis_name="s")

  @functools.partial(
      pl.kernel,
      out_type=jax.ShapeDtypeStruct((NC, NP, 8), jnp.float32),
      mesh=mesh,
      compiler_params=pltpu.CompilerParams(use_tc_tiling_on_sc=False),
      scratch_types=[
          pltpu.VMEM((CH, K), jnp.int32),
          pltpu.VMEM((K, 8), jnp.float32),
          pltpu.VMEM_SHARED((NP, 8), jnp.float32),
          pltpu.SemaphoreType.DMA,
          pltpu.SemaphoreType.DMA,
          pltpu.SemaphoreType.DMA,
          pltpu.SemaphoreType.DMA,
      ],
  )
  def degk(dst_hbm, ones_hbm, z_hbm, out_hbm, didx, onesv, acc, *sems):
    c = lax.axis_index("c")
    s = lax.axis_index("s")
    w = s * NC + c
    pltpu.sync_copy(z_hbm, acc.at[pl.ds(s * SLAB, SLAB)])
    pltpu.sync_copy(ones_hbm, onesv)
    pltpu.sync_copy(dst_hbm.at[pl.ds(w * CH, CH)], didx)
    plsc.subcore_barrier()

    for k in range(4):
      pltpu.async_copy(onesv, acc.at[didx.at[k]], sems[k], add=True)

    def body(i, carry):
      base = 4 * i
      for k in range(4):
        j = base + k
        pltpu.make_async_copy(onesv, acc.at[didx.at[j - 4]], sems[k]).wait()
        pltpu.async_copy(onesv, acc.at[didx.at[j]], sems[k], add=True)
      return carry

    lax.fori_loop(1, CH // 4, body, 0)
    for k in range(4):
      pltpu.make_async_copy(onesv, acc.at[didx.at[CH - 4 + k]],
                            sems[k]).wait()
    plsc.subcore_barrier()
    pltpu.sync_copy(acc.at[pl.ds(s * SLAB, SLAB)],
                    out_hbm.at[c, pl.ds(s * SLAB, SLAB)])

  return degk


def _dinv_of(deg_ref):
  deg = deg_ref[0, :, 0:1] + deg_ref[1, :, 0:1] + 1.0
  return lax.rsqrt(deg)


def _tc1_body(x_ref, w_ref, deg_ref, o_ref):
  dinv = _dinv_of(deg_ref)
  o_ref[...] = (jnp.dot(x_ref[...], w_ref[...],
                        preferred_element_type=jnp.float32)
                * dinv).astype(jnp.bfloat16)


def _tc2_body(sp_ref, h_ref, deg_ref, b1_ref, w2_ref, o_ref):
  dinv = _dinv_of(deg_ref)
  s = (sp_ref[0].astype(jnp.float32) + sp_ref[1].astype(jnp.float32)
       + h_ref[...].astype(jnp.float32))
  pre = s * dinv + b1_ref[...]
  h = jnp.maximum(pre, 0.0)
  o_ref[...] = jnp.dot(h, w2_ref[...],
                       preferred_element_type=jnp.float32) * dinv


def _tc3_body(s2_ref, g_ref, deg_ref, x_ref, ws_ref, bv_ref, o_ref):
  dinv = _dinv_of(deg_ref)
  s2 = (s2_ref[0].astype(jnp.float32) + s2_ref[1].astype(jnp.float32)
        + g_ref[...].astype(jnp.float32))
  o_ref[...] = (s2 * dinv
                + jnp.dot(x_ref[...], ws_ref[...],
                          preferred_element_type=jnp.float32)
                + bv_ref[...])


def kernel(x, edge_index, W1, b1, W2, b2, Ws, bs):
  N, DIN = x.shape
  DH = W1.shape[1]
  DO = W2.shape[1]
  E = edge_index.shape[1]
  f32 = jnp.float32

  NP = -(-(N + 1) // 256) * 256
  SLAB = NP // NS
  CH = -(-E // (NC * NS * K))
  CH = -(-CH // 8) * 8
  CH0 = 96
  CH1 = 2 * CH - CH0
  R = max(NC * NS * CH, (NS + 1) * CH0 + (NS - 1) * CH1)
  EP = R * K

  epi = jnp.pad(edge_index, ((0, 0), (0, EP - E)), constant_values=N)
  srcp = epi[0].reshape(R, K)
  dstp = epi[1].reshape(R, K)
  xp = jnp.pad(x, ((0, NP - N), (0, 0)))
  W2p = jnp.pad(W2, ((0, 0), (0, 8 - DO)))
  Wsp = jnp.pad(Ws, ((0, 0), (0, 8 - DO)))
  bv = jnp.pad((b2 + bs).reshape(1, DO), ((0, 0), (0, 8 - DO)))
  b1r = b1.reshape(1, DH)
  ones8 = jnp.ones((K, 8), f32)
  z_dh = jnp.zeros((SLAB, DH), jnp.bfloat16)
  z_8 = jnp.zeros((SLAB, 8), f32)
  z_8b = jnp.zeros((SLAB, 8), jnp.bfloat16)

  degp = _make_sc_degree(NP, CH)(dstp, ones8, z_8)

  BM = 1024
  grid = (NP // BM,)
  h1p = pl.pallas_call(
      _tc1_body,
      grid=grid,
      in_specs=[
          pl.BlockSpec((BM, DIN), lambda i: (i, 0)),
          pl.BlockSpec((DIN, DH), lambda i: (0, 0)),
          pl.BlockSpec((NC, BM, 8), lambda i: (0, i, 0)),
      ],
      out_specs=pl.BlockSpec((BM, DH), lambda i: (i, 0)),
      out_shape=jax.ShapeDtypeStruct((NP, DH), jnp.bfloat16),
  )(xp, W1, degp)

  Sp = _make_sc_agg(NP, DH, CH0, CH1, jnp.bfloat16)(h1p, srcp, dstp, z_dh)

  gp = pl.pallas_call(
      _tc2_body,
      grid=grid,
      in_specs=[
          pl.BlockSpec((NC, BM, DH), lambda i: (0, i, 0)),
          pl.BlockSpec((BM, DH), lambda i: (i, 0)),
          pl.BlockSpec((NC, BM, 8), lambda i: (0, i, 0)),
          pl.BlockSpec((1, DH), lambda i: (0, 0)),
          pl.BlockSpec((DH, 8), lambda i: (0, 0)),
      ],
      out_specs=pl.BlockSpec((BM, 8), lambda i: (i, 0)),
      out_shape=jax.ShapeDtypeStruct((NP, 8), f32),
  )(Sp, h1p, degp, b1r, W2p)

  S2p = _make_sc_agg(NP, 8, CH, CH, f32)(gp, srcp, dstp, z_8)

  res = pl.pallas_call(
      _tc3_body,
      grid=grid,
      in_specs=[
          pl.BlockSpec((NC, BM, 8), lambda i: (0, i, 0)),
          pl.BlockSpec((BM, 8), lambda i: (i, 0)),
          pl.BlockSpec((NC, BM, 8), lambda i: (0, i, 0)),
          pl.BlockSpec((BM, DIN), lambda i: (i, 0)),
          pl.BlockSpec((DIN, 8), lambda i: (0, 0)),
          pl.BlockSpec((1, 8), lambda i: (0, 0)),
      ],
      out_specs=pl.BlockSpec((BM, 8), lambda i: (i, 0)),
      out_shape=jax.ShapeDtypeStruct((NP, 8), f32),
  )(S2p, gp, degp, xp, Wsp, bv)

  return res[:N, :DO]


# speedup vs baseline: 1.0378x; 1.0049x over previous
import functools

import jax
import jax.numpy as jnp
from jax import lax
from jax.experimental import pallas as pl
from jax.experimental.pallas import tpu as pltpu
from jax.experimental.pallas import tpu_sc as plsc

NC = 2
NS = 16
K = 128


def _make_sc_agg(NP, W, ch0, ch1, dtype):
  SLAB = NP // NS
  D = 4
  NB = 2 * D
  CHM = max(ch0, ch1)
  mesh = plsc.VectorSubcoreMesh(core_axis_name="c", subcore_axis_name="s")

  @functools.partial(
      pl.kernel,
      out_type=jax.ShapeDtypeStruct((NC, NP, W), dtype),
      mesh=mesh,
      compiler_params=pltpu.CompilerParams(use_tc_tiling_on_sc=False),
      scratch_types=(
          [pltpu.VMEM((CHM, K), jnp.int32)] * 2
          + [pltpu.VMEM((K, W), dtype)] * NB
          + [pltpu.VMEM_SHARED((NP, W), dtype)]
          + [pltpu.SemaphoreType.DMA] * (2 * NB)
      ),
  )
  def agg(rows_hbm, src_hbm, dst_hbm, z_hbm, out_hbm, *scr):
    sidx, didx = scr[0], scr[1]
    bufs = scr[2:2 + NB]
    acc = scr[2 + NB]
    sem_g = scr[3 + NB:3 + 2 * NB]
    sem_s = scr[3 + 2 * NB:]
    c = lax.axis_index("c")
    s = lax.axis_index("s")
    pltpu.sync_copy(z_hbm, acc.at[pl.ds(s * SLAB, SLAB)])

    def start_g(t, slot):
      pltpu.async_copy(rows_hbm.at[sidx.at[t]], bufs[slot], sem_g[slot])

    def wait_g(t, slot):
      pltpu.make_async_copy(rows_hbm.at[sidx.at[t]], bufs[slot],
                            sem_g[slot]).wait()

    def start_s(t, slot):
      pltpu.async_copy(bufs[slot], acc.at[didx.at[t]], sem_s[slot], add=True)

    def wait_s(t, slot):
      pltpu.make_async_copy(bufs[slot], acc.at[didx.at[t]],
                            sem_s[slot]).wait()

    def run(ch, row0):
      pltpu.sync_copy(src_hbm.at[pl.ds(row0, ch)], sidx.at[pl.ds(0, ch)])
      pltpu.sync_copy(dst_hbm.at[pl.ds(row0, ch)], didx.at[pl.ds(0, ch)])
      plsc.subcore_barrier()
      for k in range(D):
        start_g(k, k)
      for t in range(NB):
        wait_g(t, t)
        start_s(t, t)
        if t >= D:
          wait_s(t - D, t - D)
        start_g(t + D, (t + D) % NB)

      def body(i, carry):
        base = NB * i
        for k in range(NB):
          t = base + k
          wait_g(t, k)
          start_s(t, k)
          m2 = (k + D) % NB
          wait_s(t - D, m2)
          jn = jnp.minimum(t + D, ch - 1)
          start_g(jn, m2)
        return carry

      lax.fori_loop(1, ch // NB, body, 0)
      for k in range(D):
        wait_s(ch - D + k, D + k)
      for k in range(D):
        wait_g(ch - 1, k)

    if ch0 == ch1:
      run(ch0, (s * NC + c) * ch0)
    else:
      @pl.when(c == 0)
      def _():
        run(ch0, s * ch0)

      @pl.when(c != 0)
      def _():
        run(ch1, NS * ch0 + s * ch1)

    plsc.subcore_barrier()
    pltpu.sync_copy(acc.at[pl.ds(s * SLAB, SLAB)],
                    out_hbm.at[c, pl.ds(s * SLAB, SLAB)])

  return agg


def _make_sc_degree(NP, CH):
  SLAB = NP // NS
  mesh = plsc.VectorSubcoreMesh(core_axis_name="c", subcore_axis_name="s")

  @functools.partial(
      pl.kernel,
      out_type=jax.ShapeDtypeStruct((NC, NP, 8), jnp.float32),
      mesh=mesh,
      compiler_params=pltpu.CompilerParams(use_tc_tiling_on_sc=False),
      scratch_types=[
          pltpu.VMEM((CH, K), jnp.int32),
          pltpu.VMEM((K, 8), jnp.float32),
          pltpu.VMEM_SHARED((NP, 8), jnp.float32),
          pltpu.SemaphoreType.DMA,
          pltpu.SemaphoreType.DMA,
          pltpu.SemaphoreType.DMA,
          pltpu.SemaphoreType.DMA,
      ],
  )
  def degk(dst_hbm, ones_hbm, z_hbm, out_hbm, didx, onesv, acc, *sems):
    c = lax.axis_index("c")
    s = lax.axis_index("s")
    w = s * NC + c
    pltpu.sync_copy(z_hbm, acc.at[pl.ds(s * SLAB, SLAB)])
    pltpu.sync_copy(ones_hbm, onesv)
    pltpu.sync_copy(dst_hbm.at[pl.ds(w * CH, CH)], didx)
    plsc.subcore_barrier()

    for k in range(4):
      pltpu.async_copy(onesv, acc.at[didx.at[k]], sems[k], add=True)

    def body(i, carry):
      base = 4 * i
      for k in range(4):
        j = base + k
        pltpu.make_async_copy(onesv, acc.at[didx.at[j - 4]], sems[k]).wait()
        pltpu.async_copy(onesv, acc.at[didx.at[j]], sems[k], add=True)
      return carry

    lax.fori_loop(1, CH // 4, body, 0)
    for k in range(4):
      pltpu.make_async_copy(onesv, acc.at[didx.at[CH - 4 + k]],
                            sems[k]).wait()
    plsc.subcore_barrier()
    pltpu.sync_copy(acc.at[pl.ds(s * SLAB, SLAB)],
                    out_hbm.at[c, pl.ds(s * SLAB, SLAB)])

  return degk


def _dinv_of(deg_ref):
  deg = deg_ref[0, :, 0:1] + deg_ref[1, :, 0:1] + 1.0
  return lax.rsqrt(deg)


def _tc1_body(x_ref, w_ref, deg_ref, o_ref):
  dinv = _dinv_of(deg_ref)
  o_ref[...] = (jnp.dot(x_ref[...], w_ref[...],
                        preferred_element_type=jnp.float32)
                * dinv).astype(jnp.bfloat16)


def _tc2_body(sp_ref, h_ref, deg_ref, b1_ref, w2_ref, o_ref):
  dinv = _dinv_of(deg_ref)
  s = (sp_ref[0].astype(jnp.float32) + sp_ref[1].astype(jnp.float32)
       + h_ref[...].astype(jnp.float32))
  pre = s * dinv + b1_ref[...]
  h = jnp.maximum(pre, 0.0)
  o_ref[...] = jnp.dot(h, w2_ref[...],
                       preferred_element_type=jnp.float32) * dinv


def _tc3_body(s2_ref, g_ref, deg_ref, x_ref, ws_ref, bv_ref, o_ref):
  dinv = _dinv_of(deg_ref)
  s2 = (s2_ref[0].astype(jnp.float32) + s2_ref[1].astype(jnp.float32)
        + g_ref[...].astype(jnp.float32))
  o_ref[...] = (s2 * dinv
                + jnp.dot(x_ref[...], ws_ref[...],
                          preferred_element_type=jnp.float32)
                + bv_ref[...])


def kernel(x, edge_index, W1, b1, W2, b2, Ws, bs):
  N, DIN = x.shape
  DH = W1.shape[1]
  DO = W2.shape[1]
  E = edge_index.shape[1]
  f32 = jnp.float32

  NP = -(-(N + 1) // 256) * 256
  SLAB = NP // NS
  CH = -(-E // (NC * NS * K))
  CH = -(-CH // 8) * 8
  CH0 = 96
  CH1 = 2 * CH - CH0
  R = max(NC * NS * CH, (NS + 1) * CH0 + (NS - 1) * CH1)
  EP = R * K

  src = edge_index[0]
  dst = edge_index[1]
  epad = jnp.full((EP - E,), N, dtype=jnp.int32)
  srcp = jnp.concatenate([src, epad]).reshape(R, K)
  dstp = jnp.concatenate([dst, epad]).reshape(R, K)
  xp = jnp.pad(x, ((0, NP - N), (0, 0)))
  W2p = jnp.pad(W2, ((0, 0), (0, 8 - DO)))
  Wsp = jnp.pad(Ws, ((0, 0), (0, 8 - DO)))
  bv = jnp.pad((b2 + bs).reshape(1, DO), ((0, 0), (0, 8 - DO)))
  b1r = b1.reshape(1, DH)
  ones8 = jnp.ones((K, 8), f32)
  z_dh = jnp.zeros((SLAB, DH), jnp.bfloat16)
  z_8 = jnp.zeros((SLAB, 8), f32)
  z_8b = jnp.zeros((SLAB, 8), jnp.bfloat16)

  degp = _make_sc_degree(NP, CH)(dstp, ones8, z_8)

  BM = 1024
  grid = (NP // BM,)
  h1p = pl.pallas_call(
      _tc1_body,
      grid=grid,
      in_specs=[
          pl.BlockSpec((BM, DIN), lambda i: (i, 0)),
          pl.BlockSpec((DIN, DH), lambda i: (0, 0)),
          pl.BlockSpec((NC, BM, 8), lambda i: (0, i, 0)),
      ],
      out_specs=pl.BlockSpec((BM, DH), lambda i: (i, 0)),
      out_shape=jax.ShapeDtypeStruct((NP, DH), jnp.bfloat16),
  )(xp, W1, degp)

  Sp = _make_sc_agg(NP, DH, CH0, CH1, jnp.bfloat16)(h1p, srcp, dstp, z_dh)

  gp = pl.pallas_call(
      _tc2_body,
      grid=grid,
      in_specs=[
          pl.BlockSpec((NC, BM, DH), lambda i: (0, i, 0)),
          pl.BlockSpec((BM, DH), lambda i: (i, 0)),
          pl.BlockSpec((NC, BM, 8), lambda i: (0, i, 0)),
          pl.BlockSpec((1, DH), lambda i: (0, 0)),
          pl.BlockSpec((DH, 8), lambda i: (0, 0)),
      ],
      out_specs=pl.BlockSpec((BM, 8), lambda i: (i, 0)),
      out_shape=jax.ShapeDtypeStruct((NP, 8), f32),
  )(Sp, h1p, degp, b1r, W2p)

  S2p = _make_sc_agg(NP, 8, CH, CH, f32)(gp, srcp, dstp, z_8)

  res = pl.pallas_call(
      _tc3_body,
      grid=grid,
      in_specs=[
          pl.BlockSpec((NC, BM, 8), lambda i: (0, i, 0)),
          pl.BlockSpec((BM, 8), lambda i: (i, 0)),
          pl.BlockSpec((NC, BM, 8), lambda i: (0, i, 0)),
          pl.BlockSpec((BM, DIN), lambda i: (i, 0)),
          pl.BlockSpec((DIN, 8), lambda i: (0, 0)),
          pl.BlockSpec((1, 8), lambda i: (0, 0)),
      ],
      out_specs=pl.BlockSpec((BM, 8), lambda i: (i, 0)),
      out_shape=jax.ShapeDtypeStruct((NP, 8), f32),
  )(S2p, gp, degp, xp, Wsp, bv)

  return res[:N, :DO]


# speedup vs baseline: 1.0543x; 1.0159x over previous
import functools

import jax
import jax.numpy as jnp
from jax import lax
from jax.experimental import pallas as pl
from jax.experimental.pallas import tpu as pltpu
from jax.experimental.pallas import tpu_sc as plsc

NC = 2
NS = 16
K = 128


def _make_sc_agg(NP, W, ch0, ch1, dtype):
  SLAB = NP // NS
  D = 4
  NB = 2 * D
  CHM = max(ch0, ch1)
  mesh = plsc.VectorSubcoreMesh(core_axis_name="c", subcore_axis_name="s")

  @functools.partial(
      pl.kernel,
      out_type=jax.ShapeDtypeStruct((NC, NP, W), dtype),
      mesh=mesh,
      compiler_params=pltpu.CompilerParams(use_tc_tiling_on_sc=False),
      scratch_types=(
          [pltpu.VMEM((CHM, K), jnp.int32)] * 2
          + [pltpu.VMEM((K, W), dtype)] * NB
          + [pltpu.VMEM_SHARED((NP, W), dtype)]
          + [pltpu.SemaphoreType.DMA] * (2 * NB)
      ),
  )
  def agg(rows_hbm, src_hbm, dst_hbm, z_hbm, out_hbm, *scr):
    sidx, didx = scr[0], scr[1]
    bufs = scr[2:2 + NB]
    acc = scr[2 + NB]
    sem_g = scr[3 + NB:3 + 2 * NB]
    sem_s = scr[3 + 2 * NB:]
    c = lax.axis_index("c")
    s = lax.axis_index("s")
    pltpu.sync_copy(z_hbm, acc.at[pl.ds(s * SLAB, SLAB)])

    def start_g(t, slot):
      pltpu.async_copy(rows_hbm.at[sidx.at[t]], bufs[slot], sem_g[slot])

    def wait_g(t, slot):
      pltpu.make_async_copy(rows_hbm.at[sidx.at[t]], bufs[slot],
                            sem_g[slot]).wait()

    def start_s(t, slot):
      pltpu.async_copy(bufs[slot], acc.at[didx.at[t]], sem_s[slot], add=True)

    def wait_s(t, slot):
      pltpu.make_async_copy(bufs[slot], acc.at[didx.at[t]],
                            sem_s[slot]).wait()

    def run(ch, row0):
      pltpu.sync_copy(src_hbm.at[pl.ds(row0, ch)], sidx.at[pl.ds(0, ch)])
      pltpu.sync_copy(dst_hbm.at[pl.ds(row0, ch)], didx.at[pl.ds(0, ch)])
      plsc.subcore_barrier()
      for k in range(D):
        start_g(k, k)
      for t in range(NB):
        wait_g(t, t)
        start_s(t, t)
        if t >= D:
          wait_s(t - D, t - D)
        start_g(t + D, (t + D) % NB)

      def body(i, carry):
        base = NB * i
        for k in range(NB):
          t = base + k
          wait_g(t, k)
          start_s(t, k)
          m2 = (k + D) % NB
          wait_s(t - D, m2)
          jn = jnp.minimum(t + D, ch - 1)
          start_g(jn, m2)
        return carry

      lax.fori_loop(1, ch // NB, body, 0)
      for k in range(D):
        wait_s(ch - D + k, D + k)
      for k in range(D):
        wait_g(ch - 1, k)

    if ch0 == ch1:
      run(ch0, (s * NC + c) * ch0)
    else:
      @pl.when(c == 0)
      def _():
        run(ch0, s * ch0)

      @pl.when(c != 0)
      def _():
        run(ch1, NS * ch0 + s * ch1)

    plsc.subcore_barrier()
    pltpu.sync_copy(acc.at[pl.ds(s * SLAB, SLAB)],
                    out_hbm.at[c, pl.ds(s * SLAB, SLAB)])

  return agg


def _make_sc_degree(NP, CH):
  SLAB = NP // NS
  mesh = plsc.VectorSubcoreMesh(core_axis_name="c", subcore_axis_name="s")

  @functools.partial(
      pl.kernel,
      out_type=jax.ShapeDtypeStruct((NC, NP, 8), jnp.float32),
      mesh=mesh,
      compiler_params=pltpu.CompilerParams(use_tc_tiling_on_sc=False),
      scratch_types=[
          pltpu.VMEM((CH, K), jnp.int32),
          pltpu.VMEM((K, 8), jnp.float32),
          pltpu.VMEM_SHARED((NP, 8), jnp.float32),
          pltpu.SemaphoreType.DMA,
          pltpu.SemaphoreType.DMA,
          pltpu.SemaphoreType.DMA,
          pltpu.SemaphoreType.DMA,
      ],
  )
  def degk(dst_hbm, ones_hbm, z_hbm, out_hbm, didx, onesv, acc, *sems):
    c = lax.axis_index("c")
    s = lax.axis_index("s")
    w = s * NC + c
    pltpu.sync_copy(z_hbm, acc.at[pl.ds(s * SLAB, SLAB)])
    pltpu.sync_copy(ones_hbm, onesv)
    pltpu.sync_copy(dst_hbm.at[pl.ds(w * CH, CH)], didx)
    plsc.subcore_barrier()

    for k in range(4):
      pltpu.async_copy(onesv, acc.at[didx.at[k]], sems[k], add=True)

    def body(i, carry):
      base = 4 * i
      for k in range(4):
        j = base + k
        pltpu.make_async_copy(onesv, acc.at[didx.at[j - 4]], sems[k]).wait()
        pltpu.async_copy(onesv, acc.at[didx.at[j]], sems[k], add=True)
      return carry

    lax.fori_loop(1, CH // 4, body, 0)
    for k in range(4):
      pltpu.make_async_copy(onesv, acc.at[didx.at[CH - 4 + k]],
                            sems[k]).wait()
    plsc.subcore_barrier()
    pltpu.sync_copy(acc.at[pl.ds(s * SLAB, SLAB)],
                    out_hbm.at[c, pl.ds(s * SLAB, SLAB)])

  return degk


def _dinv_of(deg_ref):
  deg = deg_ref[0, :, 0:1] + deg_ref[1, :, 0:1] + 1.0
  return lax.rsqrt(deg)


def _tc1_body(x_ref, w_ref, deg_ref, o_ref):
  dinv = _dinv_of(deg_ref)
  o_ref[...] = (jnp.dot(x_ref[...], w_ref[...],
                        preferred_element_type=jnp.float32)
                * dinv).astype(jnp.bfloat16)


def _tc2_body(sp_ref, h_ref, deg_ref, b1_ref, w2_ref, o_ref):
  dinv = _dinv_of(deg_ref)
  s = (sp_ref[0].astype(jnp.float32) + sp_ref[1].astype(jnp.float32)
       + h_ref[...].astype(jnp.float32))
  pre = s * dinv + b1_ref[...]
  h = jnp.maximum(pre, 0.0)
  o_ref[...] = jnp.dot(h, w2_ref[...],
                       preferred_element_type=jnp.float32) * dinv


def _tc3_body(s2_ref, g_ref, deg_ref, x_ref, ws_ref, bv_ref, o_ref):
  dinv = _dinv_of(deg_ref)
  s2 = (s2_ref[0].astype(jnp.float32) + s2_ref[1].astype(jnp.float32)
        + g_ref[...].astype(jnp.float32))
  o_ref[...] = (s2 * dinv
                + jnp.dot(x_ref[...], ws_ref[...],
                          preferred_element_type=jnp.float32)
                + bv_ref[...])


def kernel(x, edge_index, W1, b1, W2, b2, Ws, bs):
  N, DIN = x.shape
  DH = W1.shape[1]
  DO = W2.shape[1]
  E = edge_index.shape[1]
  f32 = jnp.float32

  NP = -(-(N + 1) // 256) * 256
  SLAB = NP // NS
  CH = -(-E // (NC * NS * K))
  CH = -(-CH // 8) * 8
  CH0 = 96
  CH1 = 2 * CH - CH0
  R = max(NC * NS * CH, (NS + 1) * CH0 + (NS - 1) * CH1)
  EP = R * K

  src = edge_index[0]
  dst = edge_index[1]
  epad = jnp.full((EP - E,), N, dtype=jnp.int32)
  srcp = jnp.concatenate([src, epad]).reshape(R, K)
  dstp = jnp.concatenate([dst, epad]).reshape(R, K)
  xp = jnp.pad(x, ((0, NP - N), (0, 0)))
  W2p = jnp.pad(W2, ((0, 0), (0, 8 - DO)))
  Wsp = jnp.pad(Ws, ((0, 0), (0, 8 - DO)))
  bv = jnp.pad((b2 + bs).reshape(1, DO), ((0, 0), (0, 8 - DO)))
  b1r = b1.reshape(1, DH)
  ones8 = jnp.ones((K, 8), f32)
  z_dh = jnp.zeros((SLAB, DH), jnp.bfloat16)
  z_8 = jnp.zeros((SLAB, 8), f32)
  z_8b = jnp.zeros((SLAB, 8), jnp.bfloat16)

  degp = _make_sc_degree(NP, CH)(dstp, ones8, z_8)

  BM = 2048
  grid = (NP // BM,)
  h1p = pl.pallas_call(
      _tc1_body,
      grid=grid,
      in_specs=[
          pl.BlockSpec((BM, DIN), lambda i: (i, 0)),
          pl.BlockSpec((DIN, DH), lambda i: (0, 0)),
          pl.BlockSpec((NC, BM, 8), lambda i: (0, i, 0)),
      ],
      out_specs=pl.BlockSpec((BM, DH), lambda i: (i, 0)),
      out_shape=jax.ShapeDtypeStruct((NP, DH), jnp.bfloat16),
  )(xp, W1, degp)

  Sp = _make_sc_agg(NP, DH, CH0, CH1, jnp.bfloat16)(h1p, srcp, dstp, z_dh)

  gp = pl.pallas_call(
      _tc2_body,
      grid=grid,
      in_specs=[
          pl.BlockSpec((NC, BM, DH), lambda i: (0, i, 0)),
          pl.BlockSpec((BM, DH), lambda i: (i, 0)),
          pl.BlockSpec((NC, BM, 8), lambda i: (0, i, 0)),
          pl.BlockSpec((1, DH), lambda i: (0, 0)),
          pl.BlockSpec((DH, 8), lambda i: (0, 0)),
      ],
      out_specs=pl.BlockSpec((BM, 8), lambda i: (i, 0)),
      out_shape=jax.ShapeDtypeStruct((NP, 8), f32),
  )(Sp, h1p, degp, b1r, W2p)

  S2p = _make_sc_agg(NP, 8, CH, CH, f32)(gp, srcp, dstp, z_8)

  res = pl.pallas_call(
      _tc3_body,
      grid=grid,
      in_specs=[
          pl.BlockSpec((NC, BM, 8), lambda i: (0, i, 0)),
          pl.BlockSpec((BM, 8), lambda i: (i, 0)),
          pl.BlockSpec((NC, BM, 8), lambda i: (0, i, 0)),
          pl.BlockSpec((BM, DIN), lambda i: (i, 0)),
          pl.BlockSpec((DIN, 8), lambda i: (0, 0)),
          pl.BlockSpec((1, 8), lambda i: (0, 0)),
      ],
      out_specs=pl.BlockSpec((BM, 8), lambda i: (i, 0)),
      out_shape=jax.ShapeDtypeStruct((NP, 8), f32),
  )(S2p, gp, degp, xp, Wsp, bv)

  return res[:N, :DO]


# speedup vs baseline: 1.0580x; 1.0034x over previous
import functools

import jax
import jax.numpy as jnp
from jax import lax
from jax.experimental import pallas as pl
from jax.experimental.pallas import tpu as pltpu
from jax.experimental.pallas import tpu_sc as plsc

NC = 2
NS = 16
K = 128


def _make_sc_agg(NP, W, ch0, ch1, dtype):
  SLAB = NP // NS
  D = 4
  NB = 2 * D
  CHM = max(ch0, ch1)
  mesh = plsc.VectorSubcoreMesh(core_axis_name="c", subcore_axis_name="s")

  @functools.partial(
      pl.kernel,
      out_type=jax.ShapeDtypeStruct((NC, NP, W), dtype),
      mesh=mesh,
      compiler_params=pltpu.CompilerParams(use_tc_tiling_on_sc=False),
      scratch_types=(
          [pltpu.VMEM((CHM, K), jnp.int32)] * 2
          + [pltpu.VMEM((K, W), dtype)] * NB
          + [pltpu.VMEM_SHARED((NP, W), dtype)]
          + [pltpu.SemaphoreType.DMA] * (2 * NB)
      ),
  )
  def agg(rows_hbm, src_hbm, dst_hbm, z_hbm, out_hbm, *scr):
    sidx, didx = scr[0], scr[1]
    bufs = scr[2:2 + NB]
    acc = scr[2 + NB]
    sem_g = scr[3 + NB:3 + 2 * NB]
    sem_s = scr[3 + 2 * NB:]
    c = lax.axis_index("c")
    s = lax.axis_index("s")
    pltpu.sync_copy(z_hbm, acc.at[pl.ds(s * SLAB, SLAB)])

    def start_g(t, slot):
      pltpu.async_copy(rows_hbm.at[sidx.at[t]], bufs[slot], sem_g[slot])

    def wait_g(t, slot):
      pltpu.make_async_copy(rows_hbm.at[sidx.at[t]], bufs[slot],
                            sem_g[slot]).wait()

    def start_s(t, slot):
      pltpu.async_copy(bufs[slot], acc.at[didx.at[t]], sem_s[slot], add=True)

    def wait_s(t, slot):
      pltpu.make_async_copy(bufs[slot], acc.at[didx.at[t]],
                            sem_s[slot]).wait()

    def run(ch, row0):
      pltpu.sync_copy(src_hbm.at[pl.ds(row0, ch)], sidx.at[pl.ds(0, ch)])
      pltpu.sync_copy(dst_hbm.at[pl.ds(row0, ch)], didx.at[pl.ds(0, ch)])
      plsc.subcore_barrier()
      for k in range(D):
        start_g(k, k)
      for t in range(NB):
        wait_g(t, t)
        start_s(t, t)
        if t >= D:
          wait_s(t - D, t - D)
        start_g(t + D, (t + D) % NB)

      def body(i, carry):
        base = NB * i
        for k in range(NB):
          t = base + k
          wait_g(t, k)
          start_s(t, k)
          m2 = (k + D) % NB
          wait_s(t - D, m2)
          jn = jnp.minimum(t + D, ch - 1)
          start_g(jn, m2)
        return carry

      lax.fori_loop(1, ch // NB, body, 0)
      for k in range(D):
        wait_s(ch - D + k, D + k)
      for k in range(D):
        wait_g(ch - 1, k)

    if ch0 == ch1:
      run(ch0, (s * NC + c) * ch0)
    else:
      @pl.when(c == 0)
      def _():
        run(ch0, s * ch0)

      @pl.when(c != 0)
      def _():
        run(ch1, NS * ch0 + s * ch1)

    plsc.subcore_barrier()
    pltpu.sync_copy(acc.at[pl.ds(s * SLAB, SLAB)],
                    out_hbm.at[c, pl.ds(s * SLAB, SLAB)])

  return agg


def _make_sc_degree(NP, CH):
  SLAB = NP // NS
  mesh = plsc.VectorSubcoreMesh(core_axis_name="c", subcore_axis_name="s")

  @functools.partial(
      pl.kernel,
      out_type=jax.ShapeDtypeStruct((NC, NP, 8), jnp.float32),
      mesh=mesh,
      compiler_params=pltpu.CompilerParams(use_tc_tiling_on_sc=False),
      scratch_types=[
          pltpu.VMEM((CH, K), jnp.int32),
          pltpu.VMEM((K, 8), jnp.float32),
          pltpu.VMEM_SHARED((NP, 8), jnp.float32),
          pltpu.SemaphoreType.DMA,
          pltpu.SemaphoreType.DMA,
          pltpu.SemaphoreType.DMA,
          pltpu.SemaphoreType.DMA,
      ],
  )
  def degk(dst_hbm, ones_hbm, z_hbm, out_hbm, didx, onesv, acc, *sems):
    c = lax.axis_index("c")
    s = lax.axis_index("s")
    w = s * NC + c
    pltpu.sync_copy(z_hbm, acc.at[pl.ds(s * SLAB, SLAB)])
    pltpu.sync_copy(ones_hbm, onesv)
    pltpu.sync_copy(dst_hbm.at[pl.ds(w * CH, CH)], didx)
    plsc.subcore_barrier()

    for k in range(4):
      pltpu.async_copy(onesv, acc.at[didx.at[k]], sems[k], add=True)

    def body(i, carry):
      base = 4 * i
      for k in range(4):
        j = base + k
        pltpu.make_async_copy(onesv, acc.at[didx.at[j - 4]], sems[k]).wait()
        pltpu.async_copy(onesv, acc.at[didx.at[j]], sems[k], add=True)
      return carry

    lax.fori_loop(1, CH // 4, body, 0)
    for k in range(4):
      pltpu.make_async_copy(onesv, acc.at[didx.at[CH - 4 + k]],
                            sems[k]).wait()
    plsc.subcore_barrier()
    pltpu.sync_copy(acc.at[pl.ds(s * SLAB, SLAB)],
                    out_hbm.at[c, pl.ds(s * SLAB, SLAB)])

  return degk


def _dinv_of(deg_ref):
  deg = deg_ref[0, :, 0:1] + deg_ref[1, :, 0:1] + 1.0
  return lax.rsqrt(deg)


def _tc1_body(x_ref, w_ref, deg_ref, o_ref):
  dinv = _dinv_of(deg_ref)
  o_ref[...] = (jnp.dot(x_ref[...], w_ref[...],
                        preferred_element_type=jnp.float32)
                * dinv).astype(jnp.bfloat16)


def _tc2_body(sp_ref, h_ref, deg_ref, b1_ref, w2_ref, o_ref):
  dinv = _dinv_of(deg_ref)
  s = (sp_ref[0].astype(jnp.float32) + sp_ref[1].astype(jnp.float32)
       + h_ref[...].astype(jnp.float32))
  pre = s * dinv + b1_ref[...]
  h = jnp.maximum(pre, 0.0)
  o_ref[...] = jnp.dot(h, w2_ref[...],
                       preferred_element_type=jnp.float32) * dinv


def _tc3_body(s2_ref, g_ref, deg_ref, x_ref, ws_ref, bv_ref, o_ref):
  dinv = _dinv_of(deg_ref)
  s2 = (s2_ref[0].astype(jnp.float32) + s2_ref[1].astype(jnp.float32)
        + g_ref[...].astype(jnp.float32))
  o_ref[...] = (s2 * dinv
                + jnp.dot(x_ref[...], ws_ref[...],
                          preferred_element_type=jnp.float32)
                + bv_ref[...])


def kernel(x, edge_index, W1, b1, W2, b2, Ws, bs):
  N, DIN = x.shape
  DH = W1.shape[1]
  DO = W2.shape[1]
  E = edge_index.shape[1]
  f32 = jnp.float32

  NP = -(-(N + 1) // 256) * 256
  SLAB = NP // NS
  CH = -(-E // (NC * NS * K))
  CH = -(-CH // 8) * 8
  CH0 = 96
  CH1 = 2 * CH - CH0
  R = max(NC * NS * CH, (NS + 1) * CH0 + (NS - 1) * CH1)
  EP = R * K

  src = edge_index[0]
  dst = edge_index[1]
  epad = jnp.full((EP - E,), N, dtype=jnp.int32)
  srcp = jnp.concatenate([src, epad]).reshape(R, K)
  dstp = jnp.concatenate([dst, epad]).reshape(R, K)
  xp = jnp.pad(x, ((0, NP - N), (0, 0)))
  W2p = jnp.pad(W2, ((0, 0), (0, 8 - DO)))
  Wsp = jnp.pad(Ws, ((0, 0), (0, 8 - DO)))
  bv = jnp.pad((b2 + bs).reshape(1, DO), ((0, 0), (0, 8 - DO)))
  b1r = b1.reshape(1, DH)
  ones8 = jnp.ones((K, 8), f32)
  z_dh = jnp.zeros((SLAB, DH), jnp.bfloat16)
  z_8 = jnp.zeros((SLAB, 8), f32)
  z_8b = jnp.zeros((SLAB, 8), jnp.bfloat16)

  degp = _make_sc_degree(NP, CH)(dstp, ones8, z_8)

  BM = 5120
  grid = (NP // BM,)
  h1p = pl.pallas_call(
      _tc1_body,
      grid=grid,
      in_specs=[
          pl.BlockSpec((BM, DIN), lambda i: (i, 0)),
          pl.BlockSpec((DIN, DH), lambda i: (0, 0)),
          pl.BlockSpec((NC, BM, 8), lambda i: (0, i, 0)),
      ],
      out_specs=pl.BlockSpec((BM, DH), lambda i: (i, 0)),
      out_shape=jax.ShapeDtypeStruct((NP, DH), jnp.bfloat16),
  )(xp, W1, degp)

  Sp = _make_sc_agg(NP, DH, CH0, CH1, jnp.bfloat16)(h1p, srcp, dstp, z_dh)

  gp = pl.pallas_call(
      _tc2_body,
      grid=grid,
      in_specs=[
          pl.BlockSpec((NC, BM, DH), lambda i: (0, i, 0)),
          pl.BlockSpec((BM, DH), lambda i: (i, 0)),
          pl.BlockSpec((NC, BM, 8), lambda i: (0, i, 0)),
          pl.BlockSpec((1, DH), lambda i: (0, 0)),
          pl.BlockSpec((DH, 8), lambda i: (0, 0)),
      ],
      out_specs=pl.BlockSpec((BM, 8), lambda i: (i, 0)),
      out_shape=jax.ShapeDtypeStruct((NP, 8), f32),
  )(Sp, h1p, degp, b1r, W2p)

  S2p = _make_sc_agg(NP, 8, CH, CH, f32)(gp, srcp, dstp, z_8)

  res = pl.pallas_call(
      _tc3_body,
      grid=grid,
      in_specs=[
          pl.BlockSpec((NC, BM, 8), lambda i: (0, i, 0)),
          pl.BlockSpec((BM, 8), lambda i: (i, 0)),
          pl.BlockSpec((NC, BM, 8), lambda i: (0, i, 0)),
          pl.BlockSpec((BM, DIN), lambda i: (i, 0)),
          pl.BlockSpec((DIN, 8), lambda i: (0, 0)),
          pl.BlockSpec((1, 8), lambda i: (0, 0)),
      ],
      out_specs=pl.BlockSpec((BM, 8), lambda i: (i, 0)),
      out_shape=jax.ShapeDtypeStruct((NP, 8), f32),
  )(S2p, gp, degp, xp, Wsp, bv)

  return res[:N, :DO]
